# initial kernel scaffold (unmeasured)
import jax
import jax.numpy as jnp
from jax import lax
from jax.experimental import pallas as pl
from jax.experimental.pallas import tpu as pltpu

N_DEV = 4
BLK = 64


def kernel(x, Wq, K_ext, V_ext, Wo):
    B, Sq, Dm = x.shape
    Skv_loc = K_ext.shape[1]
    Dh = K_ext.shape[3]
    Hl = Wq.shape[1] // Dh
    scale = 0.125

    def body(x_ref, wq_ref, k_ref, v_ref, wo_ref, out_ref,
             kbuf, vbuf, obuf,
             copy_sems, ksend, krecv, vsend, vrecv, osend, orecv):
        my = lax.axis_index("i")

        kcp = pltpu.make_async_copy(
            k_ref.at[:, :, pl.ds(my * Hl, Hl), :], kbuf.at[N_DEV - 1],
            copy_sems.at[0])
        vcp = pltpu.make_async_copy(
            v_ref.at[:, :, pl.ds(my * Hl, Hl), :], vbuf.at[N_DEV - 1],
            copy_sems.at[1])
        kcp.start()
        vcp.start()

        kv_rdmas = []
        for off in range(1, N_DEV):
            d = lax.rem(my + off, N_DEV)
            kr = pltpu.make_async_remote_copy(
                src_ref=k_ref.at[:, :, pl.ds(d * Hl, Hl), :],
                dst_ref=kbuf.at[off - 1],
                send_sem=ksend.at[off - 1], recv_sem=krecv.at[off - 1],
                device_id=(d,), device_id_type=pl.DeviceIdType.MESH)
            vr = pltpu.make_async_remote_copy(
                src_ref=v_ref.at[:, :, pl.ds(d * Hl, Hl), :],
                dst_ref=vbuf.at[off - 1],
                send_sem=vsend.at[off - 1], recv_sem=vrecv.at[off - 1],
                device_id=(d,), device_id_type=pl.DeviceIdType.MESH)
            kr.start()
            vr.start()
            kv_rdmas.append((kr, vr))

        Qs = [jnp.dot(x_ref[b], wq_ref[...], preferred_element_type=jnp.float32)
              for b in range(B)]

        masks = []
        for slot in range(N_DEV):
            src = lax.rem(my + N_DEV - 1 - slot, N_DEV)
            qb = lax.broadcasted_iota(jnp.int32, (Sq, Skv_loc), 0) // BLK
            kj = lax.broadcasted_iota(jnp.int32, (Sq, Skv_loc), 1) + src * Skv_loc
            kb = kj // BLK
            masks.append((qb == kb) | (kb == 0) | (lax.rem(qb + kb, 3) == 0))

        kcp.wait()
        vcp.wait()
        for kr, vr in kv_rdmas:
            kr.wait()
            vr.wait()

        for b in range(B):
            q_b = Qs[b].reshape(Sq, Hl, Dh)
            parts = []
            vparts = []
            for slot in range(N_DEV):
                sp = lax.dot_general(
                    q_b, kbuf[slot, b],
                    (((2,), (2,)), ((1,), (1,))),
                    preferred_element_type=jnp.float32) * scale
                parts.append(jnp.where(masks[slot][None], sp, -1e9))
                vparts.append(vbuf[slot, b])
            scores = jnp.concatenate(parts, axis=2)
            mx = jnp.max(scores, axis=2, keepdims=True)
            w = jnp.exp(scores - mx)
            w = w / jnp.sum(w, axis=2, keepdims=True)
            v_cat = jnp.concatenate(vparts, axis=0)
            ctx = lax.dot_general(
                w, v_cat, (((2,), (0,)), ((0,), (1,))),
                preferred_element_type=jnp.float32)
            ctx = ctx.transpose(1, 0, 2).reshape(Sq, Hl * Dh)
            obuf[N_DEV - 1, b] = jnp.dot(
                ctx, wo_ref[...], preferred_element_type=jnp.float32)

        o_rdmas = []
        for off in range(1, N_DEV):
            d = lax.rem(my + off, N_DEV)
            orr = pltpu.make_async_remote_copy(
                src_ref=obuf.at[N_DEV - 1],
                dst_ref=obuf.at[off - 1],
                send_sem=osend.at[off - 1], recv_sem=orecv.at[off - 1],
                device_id=(d,), device_id_type=pl.DeviceIdType.MESH)
            orr.start()
            o_rdmas.append(orr)
        for orr in o_rdmas:
            orr.wait()

        out_ref[...] = obuf[0] + obuf[1] + obuf[2] + obuf[3]

    return pl.pallas_call(
        body,
        out_shape=jax.ShapeDtypeStruct((B, Sq, Dm), jnp.float32),
        in_specs=[pl.BlockSpec(memory_space=pltpu.VMEM)] * 5,
        out_specs=pl.BlockSpec(memory_space=pltpu.VMEM),
        scratch_shapes=[
            pltpu.VMEM((N_DEV, B, Skv_loc, Hl, Dh), jnp.float32),
            pltpu.VMEM((N_DEV, B, Skv_loc, Hl, Dh), jnp.float32),
            pltpu.VMEM((N_DEV, B, Sq, Dm), jnp.float32),
            pltpu.SemaphoreType.DMA((2,)),
            pltpu.SemaphoreType.DMA((N_DEV - 1,)),
            pltpu.SemaphoreType.DMA((N_DEV - 1,)),
            pltpu.SemaphoreType.DMA((N_DEV - 1,)),
            pltpu.SemaphoreType.DMA((N_DEV - 1,)),
            pltpu.SemaphoreType.DMA((N_DEV - 1,)),
            pltpu.SemaphoreType.DMA((N_DEV - 1,)),
        ],
        compiler_params=pltpu.CompilerParams(
            vmem_limit_bytes=128 * 1024 * 1024,
        ),
    )(x, Wq, K_ext, V_ext, Wo)


# baseline (device time: 349018 ns/iter reference)
import jax
import jax.numpy as jnp
from jax import lax
from jax.experimental import pallas as pl
from jax.experimental.pallas import tpu as pltpu

N_DEV = 4
BLK = 64


def kernel(x, Wq, K_ext, V_ext, Wo):
    B, Sq, Dm = x.shape
    Skv_loc = K_ext.shape[1]
    Dh = K_ext.shape[3]
    Hl = Wq.shape[1] // Dh
    scale = 0.125

    def body(x_ref, wq_ref, k_ref, v_ref, wo_ref, out_ref,
             kbuf, vbuf, obuf,
             copy_sems, ksend, krecv, vsend, vrecv, osend, orecv):
        my = lax.axis_index("i")

        kcp = pltpu.make_async_copy(
            k_ref.at[:, :, pl.ds(my * Hl, Hl), :], kbuf.at[N_DEV - 1],
            copy_sems.at[0])
        vcp = pltpu.make_async_copy(
            v_ref.at[:, :, pl.ds(my * Hl, Hl), :], vbuf.at[N_DEV - 1],
            copy_sems.at[1])
        kcp.start()
        vcp.start()

        kv_rdmas = []
        for off in range(1, N_DEV):
            d = lax.rem(my + off, N_DEV)
            kr = pltpu.make_async_remote_copy(
                src_ref=k_ref.at[:, :, pl.ds(d * Hl, Hl), :],
                dst_ref=kbuf.at[off - 1],
                send_sem=ksend.at[off - 1], recv_sem=krecv.at[off - 1],
                device_id=(d,), device_id_type=pl.DeviceIdType.MESH)
            vr = pltpu.make_async_remote_copy(
                src_ref=v_ref.at[:, :, pl.ds(d * Hl, Hl), :],
                dst_ref=vbuf.at[off - 1],
                send_sem=vsend.at[off - 1], recv_sem=vrecv.at[off - 1],
                device_id=(d,), device_id_type=pl.DeviceIdType.MESH)
            kr.start()
            vr.start()
            kv_rdmas.append((kr, vr))

        Qs = [jnp.dot(x_ref[b], wq_ref[...], preferred_element_type=jnp.float32)
              for b in range(B)]

        masks = []
        for slot in range(N_DEV):
            src = lax.rem(my + N_DEV - 1 - slot, N_DEV)
            qb = lax.broadcasted_iota(jnp.int32, (Sq, Skv_loc), 0) // BLK
            kj = lax.broadcasted_iota(jnp.int32, (Sq, Skv_loc), 1) + src * Skv_loc
            kb = kj // BLK
            masks.append((qb == kb) | (kb == 0) | (lax.rem(qb + kb, 3) == 0))

        kcp.wait()
        vcp.wait()
        for kr, vr in kv_rdmas:
            kr.wait()
            vr.wait()

        for b in range(B):
            ctx_heads = []
            for h in range(Hl):
                q_bh = Qs[b][:, h * Dh:(h + 1) * Dh]
                parts = []
                for slot in range(N_DEV):
                    sp = lax.dot_general(
                        q_bh, kbuf[slot, b, :, h, :],
                        (((1,), (1,)), ((), ())),
                        preferred_element_type=jnp.float32) * scale
                    parts.append(jnp.where(masks[slot], sp, -1e9))
                scores = jnp.concatenate(parts, axis=1)
                mx = jnp.max(scores, axis=1, keepdims=True)
                w = jnp.exp(scores - mx)
                w = w / jnp.sum(w, axis=1, keepdims=True)
                v_cat = jnp.concatenate(
                    [vbuf[slot, b, :, h, :] for slot in range(N_DEV)], axis=0)
                ctx_heads.append(jnp.dot(
                    w, v_cat, preferred_element_type=jnp.float32))
            ctx = jnp.concatenate(ctx_heads, axis=1)
            out_ref[b] = jnp.dot(
                ctx, wo_ref[...], preferred_element_type=jnp.float32)

        o_rdmas = []
        for off in range(1, N_DEV):
            d = lax.rem(my + off, N_DEV)
            orr = pltpu.make_async_remote_copy(
                src_ref=out_ref,
                dst_ref=obuf.at[off - 1],
                send_sem=osend.at[off - 1], recv_sem=orecv.at[off - 1],
                device_id=(d,), device_id_type=pl.DeviceIdType.MESH)
            orr.start()
            o_rdmas.append(orr)
        for orr in o_rdmas:
            orr.wait()

        out_ref[...] = out_ref[...] + obuf[0] + obuf[1] + obuf[2]

    return pl.pallas_call(
        body,
        out_shape=jax.ShapeDtypeStruct((B, Sq, Dm), jnp.float32),
        in_specs=[
            pl.BlockSpec(memory_space=pltpu.VMEM),
            pl.BlockSpec(memory_space=pltpu.VMEM),
            pl.BlockSpec(memory_space=pltpu.HBM),
            pl.BlockSpec(memory_space=pltpu.HBM),
            pl.BlockSpec(memory_space=pltpu.VMEM),
        ],
        out_specs=pl.BlockSpec(memory_space=pltpu.VMEM),
        scratch_shapes=[
            pltpu.VMEM((N_DEV, B, Skv_loc, Hl, Dh), jnp.float32),
            pltpu.VMEM((N_DEV, B, Skv_loc, Hl, Dh), jnp.float32),
            pltpu.VMEM((N_DEV - 1, B, Sq, Dm), jnp.float32),
            pltpu.SemaphoreType.DMA((2,)),
            pltpu.SemaphoreType.DMA((N_DEV - 1,)),
            pltpu.SemaphoreType.DMA((N_DEV - 1,)),
            pltpu.SemaphoreType.DMA((N_DEV - 1,)),
            pltpu.SemaphoreType.DMA((N_DEV - 1,)),
            pltpu.SemaphoreType.DMA((N_DEV - 1,)),
            pltpu.SemaphoreType.DMA((N_DEV - 1,)),
        ],
        compiler_params=pltpu.CompilerParams(
            vmem_limit_bytes=64 * 1024 * 1024,
        ),
    )(x, Wq, K_ext, V_ext, Wo)
